# Initial kernel scaffold; baseline (speedup 1.0000x reference)
import jax
import jax.numpy as jnp
from jax.experimental import pallas as pl


def kernel(hidden_states, routing_weights, selected_experts, num_experts, gate_proj, up_proj, down_proj):
    raise NotImplementedError("write your pallas kernel here")


# speedup vs baseline: 1.1490x; 1.1490x over previous
import jax
import jax.numpy as jnp
from jax.experimental import pallas as pl


def _moe_body(hidden_ref, routing_ref, selected_ref, gate_ref, up_ref,
              down_ref, out_ref):
    e = pl.program_id(0)
    x = hidden_ref[...]
    g = jax.lax.dot_general(x, gate_ref[...], (((1,), (1,)), ((), ())),
                            preferred_element_type=jnp.float32)
    u = jax.lax.dot_general(x, up_ref[...], (((1,), (1,)), ((), ())),
                            preferred_element_type=jnp.float32)
    h = g * jax.nn.sigmoid(g) * u
    d = jax.lax.dot_general(h, down_ref[...], (((1,), (1,)), ((), ())),
                            preferred_element_type=jnp.float32)
    sel = selected_ref[...]
    rw = routing_ref[...]
    w = jnp.sum(jnp.where(sel == e, rw, 0.0), axis=1, keepdims=True)
    contrib = w * d

    @pl.when(e == 0)
    def _init():
        out_ref[...] = contrib

    @pl.when(e != 0)
    def _acc():
        out_ref[...] += contrib


def kernel(hidden_states, routing_weights, selected_experts, num_experts,
           gate_proj, up_proj, down_proj):
    T, H = hidden_states.shape
    K = routing_weights.shape[1]
    E, I, _ = gate_proj.shape
    return pl.pallas_call(
        _moe_body,
        grid=(E,),
        in_specs=[
            pl.BlockSpec((T, H), lambda e: (0, 0)),
            pl.BlockSpec((T, K), lambda e: (0, 0)),
            pl.BlockSpec((T, K), lambda e: (0, 0)),
            pl.BlockSpec((None, I, H), lambda e: (e, 0, 0)),
            pl.BlockSpec((None, I, H), lambda e: (e, 0, 0)),
            pl.BlockSpec((None, H, I), lambda e: (e, 0, 0)),
        ],
        out_specs=pl.BlockSpec((T, H), lambda e: (0, 0)),
        out_shape=jax.ShapeDtypeStruct((T, H), jnp.float32),
    )(hidden_states, routing_weights, selected_experts, gate_proj, up_proj,
      down_proj)


# speedup vs baseline: 1.1499x; 1.0008x over previous
import jax
import jax.numpy as jnp
from jax.experimental import pallas as pl


def _moe_body(hidden_ref, routing_ref, selected_ref, gate_ref, up_ref,
              down_ref, out_ref):
    e = pl.program_id(0)
    x = hidden_ref[...].astype(jnp.bfloat16)
    g = jax.lax.dot_general(x, gate_ref[...].astype(jnp.bfloat16),
                            (((1,), (1,)), ((), ())),
                            preferred_element_type=jnp.float32)
    u = jax.lax.dot_general(x, up_ref[...].astype(jnp.bfloat16),
                            (((1,), (1,)), ((), ())),
                            preferred_element_type=jnp.float32)
    h = g * jax.nn.sigmoid(g) * u
    d = jax.lax.dot_general(h.astype(jnp.bfloat16),
                            down_ref[...].astype(jnp.bfloat16),
                            (((1,), (1,)), ((), ())),
                            preferred_element_type=jnp.float32)
    sel = selected_ref[...]
    rw = routing_ref[...]
    w = jnp.sum(jnp.where(sel == e, rw, 0.0), axis=1, keepdims=True)
    contrib = w * d

    @pl.when(e == 0)
    def _init():
        out_ref[...] = contrib

    @pl.when(e != 0)
    def _acc():
        out_ref[...] += contrib


def kernel(hidden_states, routing_weights, selected_experts, num_experts,
           gate_proj, up_proj, down_proj):
    T, H = hidden_states.shape
    K = routing_weights.shape[1]
    E, I, _ = gate_proj.shape
    return pl.pallas_call(
        _moe_body,
        grid=(E,),
        in_specs=[
            pl.BlockSpec((T, H), lambda e: (0, 0)),
            pl.BlockSpec((T, K), lambda e: (0, 0)),
            pl.BlockSpec((T, K), lambda e: (0, 0)),
            pl.BlockSpec((None, I, H), lambda e: (e, 0, 0)),
            pl.BlockSpec((None, I, H), lambda e: (e, 0, 0)),
            pl.BlockSpec((None, H, I), lambda e: (e, 0, 0)),
        ],
        out_specs=pl.BlockSpec((T, H), lambda e: (0, 0)),
        out_shape=jax.ShapeDtypeStruct((T, H), jnp.float32),
    )(hidden_states, routing_weights, selected_experts, gate_proj, up_proj,
      down_proj)
